# baseline (speedup 1.0000x reference)
import functools

import jax
import jax.numpy as jnp
from jax import lax
from jax.experimental import pallas as pl
from jax.experimental.pallas import tpu as pltpu
from jax.experimental.pallas import tpu_sc as plsc

N_CAT = 100000
EMBED_DIM = 64
BATCH = 16384
FIELDS = 100

_INFO = plsc.get_sparse_core_info()
NC, NS = _INFO.num_cores, _INFO.num_subcores
NW = NC * NS
BW = BATCH // NW
BT_W = BW // 128
PAIRS = FIELDS // 2


def _body(obs_hbm, table_hbm, out_hbm,
          idx0, idx1, rows0, rows1, tr, isem, gsem0, gsem1, wsem):
  wid = lax.axis_index("s") * NC + lax.axis_index("c")
  b0 = wid * BW
  btg0 = wid * BT_W
  idxs = (idx0, idx1)
  rows = (rows0, rows1)
  gsems = (gsem0, gsem1)
  iota = lax.iota(jnp.int32, 16)

  def idx_src(f):
    return obs_hbm.at[f, pl.ds(b0, BW)]

  def out_dst(u):
    return out_hbm.at[u, pl.ds(0, 8), pl.ds(btg0, BT_W)]

  def tpose(src):
    def bt_body(bt, carry):
      for c in range(8):
        row16 = iota + (bt * 128 + c * 16)
        for e in range(EMBED_DIM):
          v = plsc.load_gather(src, [row16, jnp.full((16,), e, jnp.int32)])
          tr[e // 8, bt, e % 8, pl.ds(c * 16, 16)] = v
      return carry
    lax.fori_loop(0, BT_W, bt_body, 0)

  pltpu.async_copy(idx_src(0), idx0, isem)

  def pair(p, carry):
    for s in (0, 1):
      f = 2 * p + s
      cur, oth = s, 1 - s
      pltpu.make_async_copy(idx_src(f), idxs[cur], isem).wait()
      pltpu.async_copy(table_hbm.at[idxs[cur]], rows[cur], gsems[cur])

      @pl.when(f >= 1)
      def _():
        pltpu.make_async_copy(
            table_hbm.at[idxs[oth]], rows[oth], gsems[oth]).wait()

      @pl.when(f + 1 < FIELDS)
      def _():
        pltpu.async_copy(idx_src(f + 1), idxs[oth], isem)

      @pl.when(f >= 2)
      def _():
        pltpu.make_async_copy(tr, out_dst(0), wsem).wait()

      @pl.when(f >= 1)
      def _():
        tpose(rows[oth])
        pltpu.async_copy(tr, out_dst(f - 1), wsem)
    return carry

  lax.fori_loop(0, PAIRS, pair, 0)

  pltpu.make_async_copy(table_hbm.at[idxs[1]], rows[1], gsems[1]).wait()
  pltpu.make_async_copy(tr, out_dst(0), wsem).wait()
  tpose(rows[1])
  pltpu.async_copy(tr, out_dst(FIELDS - 1), wsem)
  pltpu.make_async_copy(tr, out_dst(0), wsem).wait()


@jax.jit
def kernel(obs, table):
  obs_t = obs.T.astype(jnp.int32)
  mesh = plsc.VectorSubcoreMesh(core_axis_name="c", subcore_axis_name="s")
  out5 = pl.kernel(
      _body,
      out_type=jax.ShapeDtypeStruct((FIELDS, 8, 128, 8, 128), jnp.float32),
      mesh=mesh,
      scratch_types=[
          pltpu.VMEM((BW,), jnp.int32),
          pltpu.VMEM((BW,), jnp.int32),
          pltpu.VMEM((BW, EMBED_DIM), jnp.float32),
          pltpu.VMEM((BW, EMBED_DIM), jnp.float32),
          pltpu.VMEM((8, BT_W, 8, 128), jnp.float32),
          pltpu.SemaphoreType.DMA,
          pltpu.SemaphoreType.DMA,
          pltpu.SemaphoreType.DMA,
          pltpu.SemaphoreType.DMA,
      ],
      compiler_params=pltpu.CompilerParams(
          use_tc_tiling_on_sc=False, needs_layout_passes=False),
  )(obs_t, table)
  return out5.transpose(2, 4, 0, 1, 3).reshape(BATCH, FIELDS, EMBED_DIM)


# speedup vs baseline: 1.6439x; 1.6439x over previous
import functools

import jax
import jax.numpy as jnp
from jax import lax
from jax.experimental import pallas as pl
from jax.experimental.pallas import tpu as pltpu
from jax.experimental.pallas import tpu_sc as plsc

N_CAT = 100000
EMBED_DIM = 64
BATCH = 16384
FIELDS = 100

_INFO = plsc.get_sparse_core_info()
NC, NS = _INFO.num_cores, _INFO.num_subcores
NW = NC * NS
BW = BATCH // NW
BT_W = BW // 128
PAIRS = FIELDS // 2


def _body(obs_hbm, table_hbm, out_hbm,
          idx0, idx1, rows0, rows1, tr, isem, gsem0, gsem1, wsem):
  wid = lax.axis_index("s") * NC + lax.axis_index("c")
  b0 = wid * BW
  btg0 = wid * BT_W
  idxs = (idx0, idx1)
  rows = (rows0, rows1)
  gsems = (gsem0, gsem1)
  iota = lax.iota(jnp.int32, 16)

  def idx_src(f):
    return obs_hbm.at[f, pl.ds(b0, BW)]

  def out_dst(u):
    return out_hbm.at[u, pl.ds(0, 8), pl.ds(btg0, BT_W)]

  e_vecs = tuple(jnp.full((16,), e, jnp.int32) for e in range(EMBED_DIM))

  def tpose(src):
    def bt_body(bt, carry):
      for c in range(8):
        row16 = iota + (bt * 128 + c * 16)
        for e0 in range(0, EMBED_DIM, 8):
          vs = [plsc.load_gather(src, [row16, e_vecs[e0 + k]])
                for k in range(8)]
          for k in range(8):
            tr[e0 // 8, bt, k, pl.ds(c * 16, 16)] = vs[k]
      return carry
    lax.fori_loop(0, BT_W, bt_body, 0)

  pltpu.async_copy(idx_src(0), idx0, isem)

  def pair(p, carry):
    for s in (0, 1):
      f = 2 * p + s
      cur, oth = s, 1 - s
      pltpu.make_async_copy(idx_src(f), idxs[cur], isem).wait()
      pltpu.async_copy(table_hbm.at[idxs[cur]], rows[cur], gsems[cur])

      @pl.when(f >= 1)
      def _():
        pltpu.make_async_copy(
            table_hbm.at[idxs[oth]], rows[oth], gsems[oth]).wait()

      @pl.when(f + 1 < FIELDS)
      def _():
        pltpu.async_copy(idx_src(f + 1), idxs[oth], isem)

      @pl.when(f >= 2)
      def _():
        pltpu.make_async_copy(tr, out_dst(0), wsem).wait()

      @pl.when(f >= 1)
      def _():
        tpose(rows[oth])
        pltpu.async_copy(tr, out_dst(f - 1), wsem)
    return carry

  lax.fori_loop(0, PAIRS, pair, 0)

  pltpu.make_async_copy(table_hbm.at[idxs[1]], rows[1], gsems[1]).wait()
  pltpu.make_async_copy(tr, out_dst(0), wsem).wait()
  tpose(rows[1])
  pltpu.async_copy(tr, out_dst(FIELDS - 1), wsem)
  pltpu.make_async_copy(tr, out_dst(0), wsem).wait()


@jax.jit
def kernel(obs, table):
  obs_t = obs.T.astype(jnp.int32)
  mesh = plsc.VectorSubcoreMesh(core_axis_name="c", subcore_axis_name="s")
  out5 = pl.kernel(
      _body,
      out_type=jax.ShapeDtypeStruct((FIELDS, 8, 128, 8, 128), jnp.float32),
      mesh=mesh,
      scratch_types=[
          pltpu.VMEM((BW,), jnp.int32),
          pltpu.VMEM((BW,), jnp.int32),
          pltpu.VMEM((BW, EMBED_DIM), jnp.float32),
          pltpu.VMEM((BW, EMBED_DIM), jnp.float32),
          pltpu.VMEM((8, BT_W, 8, 128), jnp.float32),
          pltpu.SemaphoreType.DMA,
          pltpu.SemaphoreType.DMA,
          pltpu.SemaphoreType.DMA,
          pltpu.SemaphoreType.DMA,
      ],
      compiler_params=pltpu.CompilerParams(
          use_tc_tiling_on_sc=False, needs_layout_passes=False,
          disable_bounds_checks=True),
  )(obs_t, table)
  return out5.transpose(2, 4, 0, 1, 3).reshape(BATCH, FIELDS, EMBED_DIM)


# speedup vs baseline: 4.0169x; 2.4435x over previous
import functools

import numpy as np

import jax
import jax.numpy as jnp
from jax import lax
from jax.experimental import pallas as pl
from jax.experimental.pallas import tpu as pltpu
from jax.experimental.pallas import tpu_sc as plsc

N_CAT = 100000
EMBED_DIM = 64
BATCH = 16384
FIELDS = 100

_INFO = plsc.get_sparse_core_info()
NC, NS = _INFO.num_cores, _INFO.num_subcores
NW = NC * NS
BW = BATCH // NW
BT_W = BW // 128
PAIRS = FIELDS // 2


def _body(obs_hbm, table_hbm, out_hbm,
          idx0, idx1, rows0, rows1, tr, isem, gsem0, gsem1, wsem):
  wid = lax.axis_index("s") * NC + lax.axis_index("c")
  b0 = wid * BW
  btg0 = wid * BT_W
  idxs = (idx0, idx1)
  rows = (rows0, rows1)
  gsems = (gsem0, gsem1)
  iota = lax.iota(jnp.int32, 16)

  def idx_src(f):
    return obs_hbm.at[f, pl.ds(b0, BW)]

  def out_dst(u):
    return out_hbm.at[u, pl.ds(0, 8), pl.ds(btg0, BT_W)]

  def tr_win():
    return tr.at[pl.ds(0, 8), pl.ds(0, BT_W), pl.ds(0, 8), pl.ds(0, 128)]

  i0c = tuple((iota + e0) // 8 for e0 in range(0, EMBED_DIM, 16))
  i2c = tuple((iota + e0) % 8 for e0 in range(0, EMBED_DIM, 16))

  def tpose(src):
    def grp_body(g, carry):
      bt = g // 8
      j0 = (g % 8) * 16
      i1 = jnp.zeros((16,), jnp.int32) + bt
      for jj in range(0, 16, 2):
        b = bt * 128 + j0 + jj
        i3a = jnp.zeros((16,), jnp.int32) + (j0 + jj)
        i3b = jnp.zeros((16,), jnp.int32) + (j0 + jj + 1)
        vsa = [src[b, pl.ds(q * 16, 16)] for q in range(4)]
        vsb = [src[b + 1, pl.ds(q * 16, 16)] for q in range(4)]
        for q in range(4):
          plsc.store_scatter(tr, [i0c[q], i1, i2c[q], i3a], vsa[q])
        for q in range(4):
          plsc.store_scatter(tr, [i0c[q], i1, i2c[q], i3b], vsb[q])
      return carry
    lax.fori_loop(0, BT_W * 8, grp_body, 0)

  pltpu.async_copy(idx_src(0), idx0, isem)

  def pair(p, carry):
    for s in (0, 1):
      f = 2 * p + s
      cur, oth = s, 1 - s
      pltpu.make_async_copy(idx_src(f), idxs[cur], isem).wait()
      pltpu.async_copy(table_hbm.at[idxs[cur]], rows[cur], gsems[cur])

      @pl.when(f >= 1)
      def _():
        pltpu.make_async_copy(
            table_hbm.at[idxs[oth]], rows[oth], gsems[oth]).wait()

      @pl.when(f + 1 < FIELDS)
      def _():
        pltpu.async_copy(idx_src(f + 1), idxs[oth], isem)

      @pl.when(f >= 2)
      def _():
        pltpu.make_async_copy(tr_win(), out_dst(0), wsem).wait()

      @pl.when(f >= 1)
      def _():
        tpose(rows[oth])
        pltpu.async_copy(tr_win(), out_dst(f - 1), wsem)
    return carry

  lax.fori_loop(0, PAIRS, pair, 0)

  pltpu.make_async_copy(table_hbm.at[idxs[1]], rows[1], gsems[1]).wait()
  pltpu.make_async_copy(tr_win(), out_dst(0), wsem).wait()
  tpose(rows[1])
  pltpu.async_copy(tr_win(), out_dst(FIELDS - 1), wsem)
  pltpu.make_async_copy(tr_win(), out_dst(0), wsem).wait()


@jax.jit
def kernel(obs, table):
  obs_t = obs.T.astype(jnp.int32)
  mesh = plsc.VectorSubcoreMesh(core_axis_name="c", subcore_axis_name="s")
  out5 = pl.kernel(
      _body,
      out_type=jax.ShapeDtypeStruct((FIELDS, 8, 128, 8, 128), jnp.float32),
      mesh=mesh,
      scratch_types=[
          pltpu.VMEM((BW,), jnp.int32),
          pltpu.VMEM((BW,), jnp.int32),
          pltpu.VMEM((BW, EMBED_DIM), jnp.float32),
          pltpu.VMEM((BW, EMBED_DIM), jnp.float32),
          pltpu.VMEM((8, BT_W, 8, 129), jnp.float32),
          pltpu.SemaphoreType.DMA,
          pltpu.SemaphoreType.DMA,
          pltpu.SemaphoreType.DMA,
          pltpu.SemaphoreType.DMA,
      ],
      compiler_params=pltpu.CompilerParams(
          use_tc_tiling_on_sc=False, needs_layout_passes=False,
          disable_bounds_checks=True),
  )(obs_t, table)
  return out5.transpose(2, 4, 0, 1, 3).reshape(BATCH, FIELDS, EMBED_DIM)


# speedup vs baseline: 4.9403x; 1.2299x over previous
import functools

import numpy as np

import jax
import jax.numpy as jnp
from jax import lax
from jax.experimental import pallas as pl
from jax.experimental.pallas import tpu as pltpu
from jax.experimental.pallas import tpu_sc as plsc

N_CAT = 100000
EMBED_DIM = 64
BATCH = 16384
FIELDS = 100

_INFO = plsc.get_sparse_core_info()
NC, NS = _INFO.num_cores, _INFO.num_subcores
NW = NC * NS
BW = BATCH // NW
HB = BW // 2
BT_U = 2
PAD = 129


def _body(obs_hbm, table_hbm, out_hbm,
          idx0, idx1, rows0, rows1, tr0, tr1,
          isem, gsem0, gsem1, wsem0, wsem1):
  wid = lax.axis_index("s") * NC + lax.axis_index("c")
  b0 = wid * BW
  btg0 = wid * (BW // 128)
  idxs = (idx0, idx1)
  rows = (rows0, rows1)
  trs = (tr0, tr1)
  gsems = (gsem0, gsem1)
  wsems = (wsem0, wsem1)
  iota = lax.iota(jnp.int32, 16)

  def idx_src(p, s):
    return obs_hbm.at[p, pl.ds(b0 + s * HB, HB)]

  def out_dst(p, s):
    return out_hbm.at[p, pl.ds(0, 8), pl.ds(btg0 + s * BT_U, BT_U)]

  def tr_win(t):
    return trs[t].at[pl.ds(0, 8), pl.ds(0, BT_U), pl.ds(0, 8), pl.ds(0, 128)]

  i0c = tuple((iota + e0) // 8 for e0 in range(0, EMBED_DIM, 16))
  i2c = tuple((iota + e0) % 8 for e0 in range(0, EMBED_DIM, 16))

  def tpose(src, t):
    tr = trs[t]

    @plsc.parallel_loop(0, BT_U * 8)
    def grp_body(g):
      bt = g // 8
      j0 = (g % 8) * 16
      i1 = jnp.zeros((16,), jnp.int32) + bt
      for jj in range(0, 16, 2):
        b = bt * 128 + j0 + jj
        i3a = jnp.zeros((16,), jnp.int32) + (j0 + jj)
        i3b = jnp.zeros((16,), jnp.int32) + (j0 + jj + 1)
        vsa = [src[b, pl.ds(q * 16, 16)] for q in range(4)]
        vsb = [src[b + 1, pl.ds(q * 16, 16)] for q in range(4)]
        for q in range(4):
          plsc.store_scatter(tr, [i0c[q], i1, i2c[q], i3a], vsa[q])
        for q in range(4):
          plsc.store_scatter(tr, [i0c[q], i1, i2c[q], i3b], vsb[q])

  pltpu.async_copy(idx_src(0, 0), idx0, isem)

  def pair(p, carry):
    for s in (0, 1):
      cur, oth = s, 1 - s
      pltpu.make_async_copy(idx_src(p, s), idxs[cur], isem).wait()
      pltpu.async_copy(table_hbm.at[idxs[cur]], rows[cur], gsems[cur])

      not_first = (p * 2 + s) >= 1
      @pl.when(not_first)
      def _():
        pltpu.make_async_copy(
            table_hbm.at[idxs[oth]], rows[oth], gsems[oth]).wait()

      if s == 0:
        pltpu.async_copy(idx_src(p, 1), idxs[oth], isem)
      else:
        @pl.when(p + 1 < FIELDS)
        def _():
          pltpu.async_copy(idx_src(p + 1, 0), idxs[oth], isem)

      @pl.when(not_first)
      def _():
        @pl.when((p * 2 + s) >= 3)
        def _():
          pltpu.make_async_copy(tr_win(oth), out_dst(0, 0), wsems[oth]).wait()
        tpose(rows[oth], oth)
        if s == 0:
          pltpu.async_copy(tr_win(oth), out_dst(p - 1, 1), wsems[oth])
        else:
          pltpu.async_copy(tr_win(oth), out_dst(p, 0), wsems[oth])
    return carry

  lax.fori_loop(0, FIELDS, pair, 0)

  pltpu.make_async_copy(table_hbm.at[idxs[1]], rows[1], gsems[1]).wait()
  pltpu.make_async_copy(tr_win(1), out_dst(0, 0), wsems[1]).wait()
  tpose(rows[1], 1)
  pltpu.async_copy(tr_win(1), out_dst(FIELDS - 1, 1), wsems[1])
  pltpu.make_async_copy(tr_win(0), out_dst(0, 0), wsems[0]).wait()
  pltpu.make_async_copy(tr_win(1), out_dst(0, 0), wsems[1]).wait()


@jax.jit
def kernel(obs, table):
  obs_t = obs.T.astype(jnp.int32)
  mesh = plsc.VectorSubcoreMesh(core_axis_name="c", subcore_axis_name="s")
  out5 = pl.kernel(
      _body,
      out_type=jax.ShapeDtypeStruct((FIELDS, 8, 128, 8, 128), jnp.float32),
      mesh=mesh,
      scratch_types=[
          pltpu.VMEM((HB,), jnp.int32),
          pltpu.VMEM((HB,), jnp.int32),
          pltpu.VMEM((HB, EMBED_DIM), jnp.float32),
          pltpu.VMEM((HB, EMBED_DIM), jnp.float32),
          pltpu.VMEM((8, BT_U, 8, PAD), jnp.float32),
          pltpu.VMEM((8, BT_U, 8, PAD), jnp.float32),
          pltpu.SemaphoreType.DMA,
          pltpu.SemaphoreType.DMA,
          pltpu.SemaphoreType.DMA,
          pltpu.SemaphoreType.DMA,
          pltpu.SemaphoreType.DMA,
      ],
      compiler_params=pltpu.CompilerParams(
          use_tc_tiling_on_sc=False, needs_layout_passes=False,
          disable_bounds_checks=True),
  )(obs_t, table)
  return out5.transpose(2, 4, 0, 1, 3).reshape(BATCH, FIELDS, EMBED_DIM)


# speedup vs baseline: 4.9802x; 1.0081x over previous
import functools

import numpy as np

import jax
import jax.numpy as jnp
from jax import lax
from jax.experimental import pallas as pl
from jax.experimental.pallas import tpu as pltpu
from jax.experimental.pallas import tpu_sc as plsc

N_CAT = 100000
EMBED_DIM = 64
BATCH = 16384
FIELDS = 100

_INFO = plsc.get_sparse_core_info()
NC, NS = _INFO.num_cores, _INFO.num_subcores
NW = NC * NS
BW = BATCH // NW
HB = BW // 2
BT_U = 2
PAD = 129


def _body(obs_hbm, table_hbm, out_hbm,
          idx_all, rows0, rows1, tr0, tr1,
          isem, gsem0, gsem1, wsem0, wsem1):
  wid = lax.axis_index("s") * NC + lax.axis_index("c")
  b0 = wid * BW
  btg0 = wid * (BW // 128)
  rows = (rows0, rows1)
  trs = (tr0, tr1)
  gsems = (gsem0, gsem1)
  wsems = (wsem0, wsem1)
  iota = lax.iota(jnp.int32, 16)

  def out_dst(p, s):
    return out_hbm.at[p, pl.ds(0, 8), pl.ds(btg0 + s * BT_U, BT_U)]

  def tr_win(t):
    return trs[t].at[pl.ds(0, 8), pl.ds(0, BT_U), pl.ds(0, 8), pl.ds(0, 128)]

  def idx_ref(p, s):
    return idx_all.at[p, pl.ds(s * HB, HB)]

  def fire_gather(p, s, buf):
    pltpu.async_copy(table_hbm.at[idx_ref(p, s)], rows[buf], gsems[buf])

  i0c = tuple((iota + e0) // 8 for e0 in range(0, EMBED_DIM, 16))
  i2c = tuple((iota + e0) % 8 for e0 in range(0, EMBED_DIM, 16))

  def tpose(src, t):
    tr = trs[t]

    @plsc.parallel_loop(0, BT_U * 8)
    def grp_body(g):
      bt = g // 8
      j0 = (g % 8) * 16
      i1 = jnp.zeros((16,), jnp.int32) + bt
      for jj in range(0, 16, 2):
        b = bt * 128 + j0 + jj
        i3a = jnp.zeros((16,), jnp.int32) + (j0 + jj)
        i3b = jnp.zeros((16,), jnp.int32) + (j0 + jj + 1)
        vsa = [src[b, pl.ds(q * 16, 16)] for q in range(4)]
        vsb = [src[b + 1, pl.ds(q * 16, 16)] for q in range(4)]
        for q in range(4):
          plsc.store_scatter(tr, [i0c[q], i1, i2c[q], i3a], vsa[q])
        for q in range(4):
          plsc.store_scatter(tr, [i0c[q], i1, i2c[q], i3b], vsb[q])

  pltpu.async_copy(obs_hbm.at[pl.ds(0, FIELDS), pl.ds(b0, BW)], idx_all,
                   isem).wait()
  fire_gather(0, 0, 0)
  fire_gather(0, 1, 1)

  def pair(p, carry):
    for s in (0, 1):
      pltpu.make_async_copy(
          table_hbm.at[idx_ref(p, s)], rows[s], gsems[s]).wait()
      @pl.when(p >= 1)
      def _():
        pltpu.make_async_copy(tr_win(s), out_dst(0, 0), wsems[s]).wait()
      tpose(rows[s], s)
      pltpu.async_copy(tr_win(s), out_dst(p, s), wsems[s])
      @pl.when(p + 1 < FIELDS)
      def _():
        fire_gather(p + 1, s, s)
    return carry

  lax.fori_loop(0, FIELDS, pair, 0)

  pltpu.make_async_copy(tr_win(0), out_dst(0, 0), wsems[0]).wait()
  pltpu.make_async_copy(tr_win(1), out_dst(0, 0), wsems[1]).wait()


@jax.jit
def kernel(obs, table):
  obs_t = obs.T.astype(jnp.int32)
  mesh = plsc.VectorSubcoreMesh(core_axis_name="c", subcore_axis_name="s")
  out5 = pl.kernel(
      _body,
      out_type=jax.ShapeDtypeStruct((FIELDS, 8, 128, 8, 128), jnp.float32),
      mesh=mesh,
      scratch_types=[
          pltpu.VMEM((FIELDS, BW), jnp.int32),
          pltpu.VMEM((HB, EMBED_DIM), jnp.float32),
          pltpu.VMEM((HB, EMBED_DIM), jnp.float32),
          pltpu.VMEM((8, BT_U, 8, PAD), jnp.float32),
          pltpu.VMEM((8, BT_U, 8, PAD), jnp.float32),
          pltpu.SemaphoreType.DMA,
          pltpu.SemaphoreType.DMA,
          pltpu.SemaphoreType.DMA,
          pltpu.SemaphoreType.DMA,
          pltpu.SemaphoreType.DMA,
      ],
      compiler_params=pltpu.CompilerParams(
          use_tc_tiling_on_sc=False, needs_layout_passes=False,
          disable_bounds_checks=True),
  )(obs_t, table)
  return out5.transpose(2, 4, 0, 1, 3).reshape(BATCH, FIELDS, EMBED_DIM)
